# baseline (speedup 1.0000x reference)
import functools

import jax
import jax.numpy as jnp
from jax import lax
from jax.experimental import pallas as pl
from jax.experimental.pallas import tpu as pltpu
from jax.experimental.pallas import tpu_sc as plsc

_NUM_CORES = 2
_NUM_SUBCORES = 16
_NUM_WORKERS = _NUM_CORES * _NUM_SUBCORES

_B = 16384
_T = 50
_D = 32
_LANES = 16

_BT_PER_W = (_B // 128) // _NUM_WORKERS
_TG = 5
_NTG = _T // _TG
_CHUNK_ROWS = _TG * 128
_N_CHUNKS = _BT_PER_W * _NTG
_TILES_PER_CHUNK = _TG * (_D // 8)
_TILE_ELEMS = 8 * 128


_N_IBLK = (1_000_000 + 127) // 128
_BLK_PER_W = (_N_IBLK + _NUM_WORKERS - 1) // _NUM_WORKERS


@functools.cache
def _make_detile():
    mesh = plsc.VectorSubcoreMesh(core_axis_name="c", subcore_axis_name="s")

    @functools.partial(
        pl.kernel,
        out_type=jax.ShapeDtypeStruct((1_000_000 * _D,), jnp.float32),
        mesh=mesh,
        scratch_types=[
            pltpu.VMEM((4, 8, 128), jnp.float32),
            pltpu.VMEM((4, 8, 128), jnp.float32),
            pltpu.VMEM((128 * _D,), jnp.float32),
            pltpu.VMEM((128 * _D,), jnp.float32),
            pltpu.SemaphoreType.DMA,
            pltpu.SemaphoreType.DMA,
            pltpu.SemaphoreType.DMA,
            pltpu.SemaphoreType.DMA,
        ],
        compiler_params=pltpu.CompilerParams(use_tc_tiling_on_sc=True,
                                             needs_layout_passes=False),
    )
    def detile_kernel(embT_hbm, out_hbm, in0, in1, tp0, tp1, g0, g1, w0, w1):
        wid = lax.axis_index("s") * _NUM_CORES + lax.axis_index("c")
        ins = (in0, in1)
        tps = (tp0, tp1)
        gsem = (g0, g1)
        wsem = (w0, w1)

        lane = jnp.arange(_LANES, dtype=jnp.int32)
        i32v = [(lane + ic * _LANES) * _D for ic in range(8)]

        def blk_of(n):
            return wid + n * _NUM_WORKERS

        def i0_of(n):
            return pl.multiple_of(blk_of(n) * 128, 128)

        def _is_last(n):
            return blk_of(n) == _N_IBLK - 1

        def start_reads(n, pb):
            i0 = i0_of(n)

            @pl.when(jnp.logical_not(_is_last(n)))
            def _full():
                pltpu.make_async_copy(
                    embT_hbm.at[:, :, pl.ds(i0, 128)], ins[pb], gsem[pb],
                ).start()

            @pl.when(_is_last(n))
            def _part():
                pltpu.make_async_copy(
                    embT_hbm.at[:, :, pl.ds(i0, 64)],
                    ins[pb].at[:, :, pl.ds(0, 64)], gsem[pb],
                ).start()

        def wait_reads(n, pb):
            @pl.when(jnp.logical_not(_is_last(n)))
            def _full():
                pltpu.make_async_copy(
                    embT_hbm.at[:, :, pl.ds(0, 128)], ins[pb], gsem[pb],
                ).wait()

            @pl.when(_is_last(n))
            def _part():
                pltpu.make_async_copy(
                    embT_hbm.at[:, :, pl.ds(0, 64)],
                    ins[pb].at[:, :, pl.ds(0, 64)], gsem[pb],
                ).wait()

        def transpose(pb):
            @pl.loop(0, 4)
            def _mo(mo):
                for mi in range(8):
                    m = mo * 8 + mi
                    j = ((lane + m) & 15) + (m >> 4) * 16
                    jt = j >> 3
                    jl = j & 7
                    for ic in range(8):
                        i = lane + ic * _LANES
                        vals = plsc.load_gather(ins[pb], [jt, jl, i])
                        plsc.store_scatter(tps[pb], [i32v[ic] + j], vals)

        def start_write(n, pb):
            @pl.when(jnp.logical_not(_is_last(n)))
            def _full():
                pltpu.make_async_copy(
                    tps[pb], out_hbm.at[pl.ds(i0_of(n) * _D, 128 * _D)],
                    wsem[pb],
                ).start()

            @pl.when(_is_last(n))
            def _part():
                pltpu.make_async_copy(
                    tps[pb].at[pl.ds(0, 64 * _D)],
                    out_hbm.at[pl.ds(i0_of(n) * _D, 64 * _D)], wsem[pb],
                ).start()

        def wait_write(n, pb):
            @pl.when(jnp.logical_not(_is_last(n)))
            def _full():
                pltpu.make_async_copy(
                    tps[pb], out_hbm.at[pl.ds(i0_of(n) * _D, 128 * _D)],
                    wsem[pb],
                ).wait()

            @pl.when(_is_last(n))
            def _part():
                pltpu.make_async_copy(
                    tps[pb].at[pl.ds(0, 64 * _D)],
                    out_hbm.at[pl.ds(i0_of(n) * _D, 64 * _D)], wsem[pb],
                ).wait()

        def valid(n):
            return blk_of(n) < _N_IBLK

        for n0 in range(2):
            @pl.when(valid(n0))
            def _pre():
                start_reads(n0, n0)

        @pl.loop(0, (_BLK_PER_W + 1) // 2)
        def _pair(p):
            for half in range(2):
                n = 2 * p + half
                pb = half

                @pl.when(valid(n))
                def _do():
                    wait_reads(n, pb)

                    @pl.when(n >= 2)
                    def _w():
                        wait_write(n - 2, pb)

                    transpose(pb)
                    start_write(n, pb)

                    @pl.when(valid(n + 2))
                    def _pf():
                        start_reads(n + 2, pb)

        for nt in (_BLK_PER_W - 2, _BLK_PER_W - 1):
            @pl.when(valid(nt))
            def _drain():
                wait_write(nt, nt % 2)

    return detile_kernel


@functools.cache
def _make_kernel():
    n_out = _B * _T * _D
    idx_per_w = _BT_PER_W * 128 * _T
    mesh = plsc.VectorSubcoreMesh(core_axis_name="c", subcore_axis_name="s")

    @functools.partial(
        pl.kernel,
        out_type=jax.ShapeDtypeStruct((n_out,), jnp.float32),
        mesh=mesh,
        scratch_types=[
            pltpu.VMEM((idx_per_w,), jnp.int32),
            pltpu.VMEM((_CHUNK_ROWS,), jnp.int32),
            pltpu.VMEM((_CHUNK_ROWS,), jnp.int32),
            pltpu.VMEM((_CHUNK_ROWS, _D), jnp.float32),
            pltpu.VMEM((_CHUNK_ROWS, _D), jnp.float32),
            pltpu.VMEM((_TILES_PER_CHUNK * _TILE_ELEMS,), jnp.float32),
            pltpu.VMEM((_TILES_PER_CHUNK * _TILE_ELEMS,), jnp.float32),
            pltpu.SemaphoreType.DMA,
            pltpu.SemaphoreType.DMA,
            pltpu.SemaphoreType.DMA,
            pltpu.SemaphoreType.DMA,
        ],
        compiler_params=pltpu.CompilerParams(use_tc_tiling_on_sc=False,
                                             needs_layout_passes=False),
    )
    def gather_kernel(idx_hbm, table_hbm, out_hbm,
                      idx_all, ord0, ord1, rows0, rows1, tiles0, tiles1,
                      g0, g1, w0, w1):
        wid = lax.axis_index("s") * _NUM_CORES + lax.axis_index("c")
        ords = (ord0, ord1)
        rows = (rows0, rows1)
        tiles = (tiles0, tiles1)
        gsem = (g0, g1)
        wsem = (w0, w1)
        bt0 = wid * _BT_PER_W

        pltpu.sync_copy(idx_hbm.at[pl.ds(wid * idx_per_w, idx_per_w)], idx_all)

        lane = jnp.arange(_LANES, dtype=jnp.int32)
        bc_vecs = [lane + bc * _LANES for bc in range(128 // _LANES)]
        diag = [(lane + c0) & 7 for c0 in range(8)]
        sdiag = [d * 128 + lane for d in diag]

        def build_ord(c, pb):
            blk_off = (c // _NTG) * (128 * _T)
            t0 = (c % _NTG) * _TG

            @pl.loop(0, _TG)
            def _tl(tl):
                t_abs = t0 + tl
                for bc in range(128 // _LANES):
                    pos = blk_off + bc_vecs[bc] * _T + t_abs
                    vals = plsc.load_gather(idx_all, [pos])
                    ords[pb][pl.ds(tl * 128 + bc * _LANES, _LANES)] = vals

        def start_gather(pb):
            pltpu.make_async_copy(table_hbm.at[ords[pb]], rows[pb],
                                  gsem[pb]).start()

        def wait_gather(pb):
            pltpu.make_async_copy(table_hbm.at[ords[pb]], rows[pb],
                                  gsem[pb]).wait()

        def rearrange(pb):
            @pl.loop(0, _TG)
            def _tl(tl):
                rowv = [bc_vecs[bc] + tl * 128 for bc in range(128 // _LANES)]
                for dt in range(_D // 8):
                    tbase = (tl * 4 + dt) * _TILE_ELEMS
                    for c0 in range(8):
                        col = diag[c0] + dt * 8
                        sdc = sdiag[c0] + tbase
                        for bc in range(128 // _LANES):
                            vals = plsc.load_gather(rows[pb], [rowv[bc], col])
                            plsc.store_scatter(tiles[pb],
                                               [sdc + bc * _LANES], vals)

        def fire_tile_writes(c, pb):
            bt_abs = bt0 + c // _NTG
            t0 = (c % _NTG) * _TG
            for k in range(_TILES_PER_CHUNK):
                tl, dt = k // 4, k % 4
                trow = (t0 + tl) * 512 + dt * 128 + bt_abs
                pltpu.make_async_copy(
                    tiles[pb].at[pl.ds(k * _TILE_ELEMS, _TILE_ELEMS)],
                    out_hbm.at[pl.ds(trow * _TILE_ELEMS, _TILE_ELEMS)],
                    wsem[pb],
                ).start()

        def wait_tile_writes(pb):
            pltpu.make_async_copy(
                tiles[pb],
                out_hbm.at[pl.ds(0, _TILES_PER_CHUNK * _TILE_ELEMS)],
                wsem[pb],
            ).wait()

        build_ord(0, 0)
        start_gather(0)

        @pl.loop(0, _N_CHUNKS // 2)
        def _pair(p):
            for half in range(2):
                c = 2 * p + half
                pb = half
                wait_gather(pb)

                @pl.when(c + 1 < _N_CHUNKS)
                def _prefetch():
                    build_ord(c + 1, 1 - pb)
                    start_gather(1 - pb)

                @pl.when(c >= 2)
                def _free_buf():
                    wait_tile_writes(pb)

                rearrange(pb)
                fire_tile_writes(c, pb)

        wait_tile_writes(0)
        wait_tile_writes(1)

    return gather_kernel


def kernel(token_ids, embeddings):
    idx = token_ids.reshape(-1).astype(jnp.int32)
    table_rm = _make_detile()(embeddings.T.reshape(4, 8, 1_000_000))
    table_rm = table_rm.reshape(1_000_000, _D)
    out = _make_kernel()(idx, table_rm)
    return (out.reshape(_T, 4, 128, 8, 128)
               .transpose(2, 4, 0, 1, 3)
               .reshape(_B, _T, _D))


# speedup vs baseline: 1.0326x; 1.0326x over previous
import functools

import jax
import jax.numpy as jnp
from jax import lax
from jax.experimental import pallas as pl
from jax.experimental.pallas import tpu as pltpu
from jax.experimental.pallas import tpu_sc as plsc

_NUM_CORES = 2
_NUM_SUBCORES = 16
_NUM_WORKERS = _NUM_CORES * _NUM_SUBCORES

_B = 16384
_T = 50
_D = 32
_LANES = 16

_BT_PER_W = (_B // 128) // _NUM_WORKERS
_TG = 5
_NTG = _T // _TG
_CHUNK_ROWS = _TG * 128
_N_CHUNKS = _BT_PER_W * _NTG
_TILES_PER_CHUNK = _TG * (_D // 8)
_TILE_ELEMS = 8 * 128


_N_IBLK = (1_000_000 + 127) // 128
_BLK_PER_W = (_N_IBLK + _NUM_WORKERS - 1) // _NUM_WORKERS


@functools.cache
def _make_detile():
    mesh = plsc.VectorSubcoreMesh(core_axis_name="c", subcore_axis_name="s")

    @functools.partial(
        pl.kernel,
        out_type=jax.ShapeDtypeStruct((1_000_000 * _D,), jnp.float32),
        mesh=mesh,
        scratch_types=[
            pltpu.VMEM((4, 8, 128), jnp.float32),
            pltpu.VMEM((4, 8, 128), jnp.float32),
            pltpu.VMEM((128 * _D,), jnp.float32),
            pltpu.VMEM((128 * _D,), jnp.float32),
            pltpu.SemaphoreType.DMA,
            pltpu.SemaphoreType.DMA,
            pltpu.SemaphoreType.DMA,
            pltpu.SemaphoreType.DMA,
        ],
        compiler_params=pltpu.CompilerParams(use_tc_tiling_on_sc=True,
                                             needs_layout_passes=False),
    )
    def detile_kernel(embT_hbm, out_hbm, in0, in1, tp0, tp1, g0, g1, w0, w1):
        wid = lax.axis_index("s") * _NUM_CORES + lax.axis_index("c")
        ins = (in0, in1)
        tps = (tp0, tp1)
        gsem = (g0, g1)
        wsem = (w0, w1)

        lane = jnp.arange(_LANES, dtype=jnp.int32)
        i32v = [(lane + ic * _LANES) * _D for ic in range(8)]

        def blk_of(n):
            return wid + n * _NUM_WORKERS

        def i0_of(n):
            return pl.multiple_of(blk_of(n) * 128, 128)

        def _is_last(n):
            return blk_of(n) == _N_IBLK - 1

        def start_reads(n, pb):
            i0 = i0_of(n)

            @pl.when(jnp.logical_not(_is_last(n)))
            def _full():
                pltpu.make_async_copy(
                    embT_hbm.at[:, :, pl.ds(i0, 128)], ins[pb], gsem[pb],
                ).start()

            @pl.when(_is_last(n))
            def _part():
                pltpu.make_async_copy(
                    embT_hbm.at[:, :, pl.ds(i0, 64)],
                    ins[pb].at[:, :, pl.ds(0, 64)], gsem[pb],
                ).start()

        def wait_reads(n, pb):
            @pl.when(jnp.logical_not(_is_last(n)))
            def _full():
                pltpu.make_async_copy(
                    embT_hbm.at[:, :, pl.ds(0, 128)], ins[pb], gsem[pb],
                ).wait()

            @pl.when(_is_last(n))
            def _part():
                pltpu.make_async_copy(
                    embT_hbm.at[:, :, pl.ds(0, 64)],
                    ins[pb].at[:, :, pl.ds(0, 64)], gsem[pb],
                ).wait()

        def transpose(pb):
            @pl.loop(0, 4)
            def _mo(mo):
                for mi in range(8):
                    m = mo * 8 + mi
                    j = ((lane + m) & 15) + (m >> 4) * 16
                    jt = j >> 3
                    jl = j & 7
                    for ic in range(8):
                        i = lane + ic * _LANES
                        vals = plsc.load_gather(ins[pb], [jt, jl, i])
                        plsc.store_scatter(tps[pb], [i32v[ic] + j], vals)

        def start_write(n, pb):
            @pl.when(jnp.logical_not(_is_last(n)))
            def _full():
                pltpu.make_async_copy(
                    tps[pb], out_hbm.at[pl.ds(i0_of(n) * _D, 128 * _D)],
                    wsem[pb],
                ).start()

            @pl.when(_is_last(n))
            def _part():
                pltpu.make_async_copy(
                    tps[pb].at[pl.ds(0, 64 * _D)],
                    out_hbm.at[pl.ds(i0_of(n) * _D, 64 * _D)], wsem[pb],
                ).start()

        def wait_write(n, pb):
            @pl.when(jnp.logical_not(_is_last(n)))
            def _full():
                pltpu.make_async_copy(
                    tps[pb], out_hbm.at[pl.ds(i0_of(n) * _D, 128 * _D)],
                    wsem[pb],
                ).wait()

            @pl.when(_is_last(n))
            def _part():
                pltpu.make_async_copy(
                    tps[pb].at[pl.ds(0, 64 * _D)],
                    out_hbm.at[pl.ds(i0_of(n) * _D, 64 * _D)], wsem[pb],
                ).wait()

        def valid(n):
            return blk_of(n) < _N_IBLK

        for n0 in range(2):
            @pl.when(valid(n0))
            def _pre():
                start_reads(n0, n0)

        @pl.loop(0, (_BLK_PER_W + 1) // 2)
        def _pair(p):
            for half in range(2):
                n = 2 * p + half
                pb = half

                @pl.when(valid(n))
                def _do():
                    wait_reads(n, pb)

                    @pl.when(n >= 2)
                    def _w():
                        wait_write(n - 2, pb)

                    transpose(pb)
                    start_write(n, pb)

                    @pl.when(valid(n + 2))
                    def _pf():
                        start_reads(n + 2, pb)

        for nt in (_BLK_PER_W - 2, _BLK_PER_W - 1):
            @pl.when(valid(nt))
            def _drain():
                wait_write(nt, nt % 2)

    return detile_kernel


@functools.cache
def _make_kernel():
    n_out = _B * _T * _D
    idx_per_w = _BT_PER_W * 128 * _T
    mesh = plsc.VectorSubcoreMesh(core_axis_name="c", subcore_axis_name="s")

    @functools.partial(
        pl.kernel,
        out_type=jax.ShapeDtypeStruct((n_out,), jnp.float32),
        mesh=mesh,
        scratch_types=[
            pltpu.VMEM((idx_per_w,), jnp.int32),
            pltpu.VMEM((_CHUNK_ROWS,), jnp.int32),
            pltpu.VMEM((_CHUNK_ROWS,), jnp.int32),
            pltpu.VMEM((_CHUNK_ROWS, _D), jnp.float32),
            pltpu.VMEM((_CHUNK_ROWS, _D), jnp.float32),
            pltpu.VMEM((_TILES_PER_CHUNK * _TILE_ELEMS,), jnp.float32),
            pltpu.VMEM((_TILES_PER_CHUNK * _TILE_ELEMS,), jnp.float32),
            pltpu.SemaphoreType.DMA,
            pltpu.SemaphoreType.DMA,
            pltpu.SemaphoreType.DMA,
            pltpu.SemaphoreType.DMA,
        ],
        compiler_params=pltpu.CompilerParams(use_tc_tiling_on_sc=False,
                                             needs_layout_passes=False),
    )
    def gather_kernel(idx_hbm, table_hbm, out_hbm,
                      idx_all, ord0, ord1, rows0, rows1, tiles0, tiles1,
                      g0, g1, w0, w1):
        wid = lax.axis_index("s") * _NUM_CORES + lax.axis_index("c")
        ords = (ord0, ord1)
        rows = (rows0, rows1)
        tiles = (tiles0, tiles1)
        gsem = (g0, g1)
        wsem = (w0, w1)
        bt0 = wid * _BT_PER_W

        pltpu.sync_copy(idx_hbm.at[pl.ds(wid * idx_per_w, idx_per_w)], idx_all)

        lane = jnp.arange(_LANES, dtype=jnp.int32)
        bc_vecs = [lane + bc * _LANES for bc in range(128 // _LANES)]
        diag = [(lane + c0) & 7 for c0 in range(8)]
        sdiag = [d * 128 + lane for d in diag]

        def build_ord(c, pb):
            blk_off = (c // _NTG) * (128 * _T)
            t0 = (c % _NTG) * _TG

            @pl.loop(0, _TG)
            def _tl(tl):
                t_abs = t0 + tl
                for bc in range(128 // _LANES):
                    pos = blk_off + bc_vecs[bc] * _T + t_abs
                    vals = plsc.load_gather(idx_all, [pos])
                    ords[pb][pl.ds(tl * 128 + bc * _LANES, _LANES)] = vals

        def start_gather(pb):
            pltpu.make_async_copy(table_hbm.at[ords[pb]], rows[pb],
                                  gsem[pb]).start()

        def wait_gather(pb):
            pltpu.make_async_copy(table_hbm.at[ords[pb]], rows[pb],
                                  gsem[pb]).wait()

        def rearrange(pb):
            @pl.loop(0, _TG)
            def _tl(tl):
                for bc in range(128 // _LANES):
                    row_vec = bc_vecs[bc] + tl * 128
                    for dt in range(_D // 8):
                        tbase = (tl * 4 + dt) * _TILE_ELEMS + bc * _LANES
                        for c0 in range(8):
                            col = diag[c0] + dt * 8
                            vals = plsc.load_gather(rows[pb], [row_vec, col])
                            plsc.store_scatter(tiles[pb], [sdiag[c0] + tbase],
                                               vals)

        def fire_tile_writes(c, pb):
            bt_abs = bt0 + c // _NTG
            t0 = (c % _NTG) * _TG
            for k in range(_TILES_PER_CHUNK):
                tl, dt = k // 4, k % 4
                trow = (t0 + tl) * 512 + dt * 128 + bt_abs
                pltpu.make_async_copy(
                    tiles[pb].at[pl.ds(k * _TILE_ELEMS, _TILE_ELEMS)],
                    out_hbm.at[pl.ds(trow * _TILE_ELEMS, _TILE_ELEMS)],
                    wsem[pb],
                ).start()

        def wait_tile_writes(pb):
            pltpu.make_async_copy(
                tiles[pb],
                out_hbm.at[pl.ds(0, _TILES_PER_CHUNK * _TILE_ELEMS)],
                wsem[pb],
            ).wait()

        build_ord(0, 0)
        start_gather(0)

        @pl.loop(0, _N_CHUNKS // 2)
        def _pair(p):
            for half in range(2):
                c = 2 * p + half
                pb = half
                wait_gather(pb)

                @pl.when(c + 1 < _N_CHUNKS)
                def _prefetch():
                    build_ord(c + 1, 1 - pb)
                    start_gather(1 - pb)

                @pl.when(c >= 2)
                def _free_buf():
                    wait_tile_writes(pb)

                rearrange(pb)
                fire_tile_writes(c, pb)

        wait_tile_writes(0)
        wait_tile_writes(1)

    return gather_kernel


def kernel(token_ids, embeddings):
    idx = token_ids.reshape(-1).astype(jnp.int32)
    table_rm = _make_detile()(embeddings.T.reshape(4, 8, 1_000_000))
    table_rm = table_rm.reshape(1_000_000, _D)
    out = _make_kernel()(idx, table_rm)
    return (out.reshape(_T, 4, 128, 8, 128)
               .transpose(2, 4, 0, 1, 3)
               .reshape(_B, _T, _D))


# speedup vs baseline: 1.2059x; 1.1678x over previous
import functools

import jax
import jax.numpy as jnp
from jax import lax
from jax.experimental import pallas as pl
from jax.experimental.pallas import tpu as pltpu
from jax.experimental.pallas import tpu_sc as plsc

_NUM_CORES = 2
_NUM_SUBCORES = 16
_NUM_WORKERS = _NUM_CORES * _NUM_SUBCORES

_B = 16384
_T = 50
_D = 32
_LANES = 16

_BT_PER_W = (_B // 128) // _NUM_WORKERS
_TG = 5
_NTG = _T // _TG
_CHUNK_ROWS = _TG * 128
_N_CHUNKS = _BT_PER_W * _NTG
_TILES_PER_CHUNK = _TG * (_D // 8)
_TILE_ELEMS = 8 * 128


_N_IBLK = (1_000_000 + 127) // 128
_BLK_PER_W = (_N_IBLK + _NUM_WORKERS - 1) // _NUM_WORKERS


@functools.cache
def _make_detile():
    mesh = plsc.VectorSubcoreMesh(core_axis_name="c", subcore_axis_name="s")

    @functools.partial(
        pl.kernel,
        out_type=jax.ShapeDtypeStruct((1_000_000 * _D,), jnp.float32),
        mesh=mesh,
        scratch_types=[
            pltpu.VMEM((4, 8, 128), jnp.float32),
            pltpu.VMEM((4, 8, 128), jnp.float32),
            pltpu.VMEM((128 * _D,), jnp.float32),
            pltpu.VMEM((128 * _D,), jnp.float32),
            pltpu.SemaphoreType.DMA,
            pltpu.SemaphoreType.DMA,
            pltpu.SemaphoreType.DMA,
            pltpu.SemaphoreType.DMA,
        ],
        compiler_params=pltpu.CompilerParams(use_tc_tiling_on_sc=True,
                                             needs_layout_passes=False),
    )
    def detile_kernel(embT_hbm, out_hbm, in0, in1, tp0, tp1, g0, g1, w0, w1):
        wid = lax.axis_index("s") * _NUM_CORES + lax.axis_index("c")
        ins = (in0, in1)
        tps = (tp0, tp1)
        gsem = (g0, g1)
        wsem = (w0, w1)

        lane = jnp.arange(_LANES, dtype=jnp.int32)
        i32v = [(lane + ic * _LANES) * _D for ic in range(8)]

        def blk_of(n):
            return wid + n * _NUM_WORKERS

        def i0_of(n):
            return pl.multiple_of(blk_of(n) * 128, 128)

        def _is_last(n):
            return blk_of(n) == _N_IBLK - 1

        def start_reads(n, pb):
            i0 = i0_of(n)

            @pl.when(jnp.logical_not(_is_last(n)))
            def _full():
                pltpu.make_async_copy(
                    embT_hbm.at[:, :, pl.ds(i0, 128)], ins[pb], gsem[pb],
                ).start()

            @pl.when(_is_last(n))
            def _part():
                pltpu.make_async_copy(
                    embT_hbm.at[:, :, pl.ds(i0, 64)],
                    ins[pb].at[:, :, pl.ds(0, 64)], gsem[pb],
                ).start()

        def wait_reads(n, pb):
            @pl.when(jnp.logical_not(_is_last(n)))
            def _full():
                pltpu.make_async_copy(
                    embT_hbm.at[:, :, pl.ds(0, 128)], ins[pb], gsem[pb],
                ).wait()

            @pl.when(_is_last(n))
            def _part():
                pltpu.make_async_copy(
                    embT_hbm.at[:, :, pl.ds(0, 64)],
                    ins[pb].at[:, :, pl.ds(0, 64)], gsem[pb],
                ).wait()

        def transpose(pb):
            @pl.loop(0, 32)
            def _m(m):
                j = ((lane + m) & 15) + (m >> 4) * 16
                jt = j >> 3
                jl = j & 7
                for ic in range(8):
                    i = lane + ic * _LANES
                    vals = plsc.load_gather(ins[pb], [jt, jl, i])
                    plsc.store_scatter(tps[pb], [i32v[ic] + j], vals)

        def start_write(n, pb):
            @pl.when(jnp.logical_not(_is_last(n)))
            def _full():
                pltpu.make_async_copy(
                    tps[pb], out_hbm.at[pl.ds(i0_of(n) * _D, 128 * _D)],
                    wsem[pb],
                ).start()

            @pl.when(_is_last(n))
            def _part():
                pltpu.make_async_copy(
                    tps[pb].at[pl.ds(0, 64 * _D)],
                    out_hbm.at[pl.ds(i0_of(n) * _D, 64 * _D)], wsem[pb],
                ).start()

        def wait_write(n, pb):
            @pl.when(jnp.logical_not(_is_last(n)))
            def _full():
                pltpu.make_async_copy(
                    tps[pb], out_hbm.at[pl.ds(i0_of(n) * _D, 128 * _D)],
                    wsem[pb],
                ).wait()

            @pl.when(_is_last(n))
            def _part():
                pltpu.make_async_copy(
                    tps[pb].at[pl.ds(0, 64 * _D)],
                    out_hbm.at[pl.ds(i0_of(n) * _D, 64 * _D)], wsem[pb],
                ).wait()

        def valid(n):
            return blk_of(n) < _N_IBLK

        for n0 in range(2):
            @pl.when(valid(n0))
            def _pre():
                start_reads(n0, n0)

        @pl.loop(0, (_BLK_PER_W + 1) // 2)
        def _pair(p):
            for half in range(2):
                n = 2 * p + half
                pb = half

                @pl.when(valid(n))
                def _do():
                    wait_reads(n, pb)

                    @pl.when(n >= 2)
                    def _w():
                        wait_write(n - 2, pb)

                    transpose(pb)
                    start_write(n, pb)

                    @pl.when(valid(n + 2))
                    def _pf():
                        start_reads(n + 2, pb)

        for nt in (_BLK_PER_W - 2, _BLK_PER_W - 1):
            @pl.when(valid(nt))
            def _drain():
                wait_write(nt, nt % 2)

    return detile_kernel


@functools.cache
def _make_kernel():
    n_out = _B * _T * _D
    idx_per_w = _BT_PER_W * 128 * _T
    mesh = plsc.VectorSubcoreMesh(core_axis_name="c", subcore_axis_name="s")

    @functools.partial(
        pl.kernel,
        out_type=jax.ShapeDtypeStruct((n_out,), jnp.float32),
        mesh=mesh,
        scratch_types=[
            pltpu.VMEM((idx_per_w,), jnp.int32),
            pltpu.VMEM((_CHUNK_ROWS,), jnp.int32),
            pltpu.VMEM((_CHUNK_ROWS,), jnp.int32),
            pltpu.VMEM((_CHUNK_ROWS, _D), jnp.float32),
            pltpu.VMEM((_CHUNK_ROWS, _D), jnp.float32),
            pltpu.VMEM((_TILES_PER_CHUNK * _TILE_ELEMS,), jnp.float32),
            pltpu.VMEM((_TILES_PER_CHUNK * _TILE_ELEMS,), jnp.float32),
            pltpu.SemaphoreType.DMA,
            pltpu.SemaphoreType.DMA,
            pltpu.SemaphoreType.DMA,
            pltpu.SemaphoreType.DMA,
        ],
        compiler_params=pltpu.CompilerParams(use_tc_tiling_on_sc=False,
                                             needs_layout_passes=False),
    )
    def gather_kernel(idx_hbm, table_hbm, out_hbm,
                      idx_all, ord0, ord1, rows0, rows1, tiles0, tiles1,
                      g0, g1, w0, w1):
        wid = lax.axis_index("s") * _NUM_CORES + lax.axis_index("c")
        ords = (ord0, ord1)
        rows = (rows0, rows1)
        tiles = (tiles0, tiles1)
        gsem = (g0, g1)
        wsem = (w0, w1)
        bt0 = wid * _BT_PER_W

        pltpu.sync_copy(idx_hbm.at[pl.ds(wid * idx_per_w, idx_per_w)], idx_all)

        lane = jnp.arange(_LANES, dtype=jnp.int32)
        bc_vecs = [lane + bc * _LANES for bc in range(128 // _LANES)]
        diag = [(lane + c0) & 7 for c0 in range(8)]
        sdiag = [d * 128 + lane for d in diag]

        def build_ord(c, pb):
            blk_off = (c // _NTG) * (128 * _T)
            t0 = (c % _NTG) * _TG

            @pl.loop(0, _TG)
            def _tl(tl):
                t_abs = t0 + tl
                for bc in range(128 // _LANES):
                    pos = blk_off + bc_vecs[bc] * _T + t_abs
                    vals = plsc.load_gather(idx_all, [pos])
                    ords[pb][pl.ds(tl * 128 + bc * _LANES, _LANES)] = vals

        def start_gather(pb):
            pltpu.make_async_copy(table_hbm.at[ords[pb]], rows[pb],
                                  gsem[pb]).start()

        def wait_gather(pb):
            pltpu.make_async_copy(table_hbm.at[ords[pb]], rows[pb],
                                  gsem[pb]).wait()

        def rearrange(pb):
            @pl.loop(0, _TG)
            def _tl(tl):
                for bc in range(128 // _LANES):
                    row_vec = bc_vecs[bc] + tl * 128
                    for dt in range(_D // 8):
                        tbase = (tl * 4 + dt) * _TILE_ELEMS + bc * _LANES
                        for c0 in range(8):
                            col = diag[c0] + dt * 8
                            vals = plsc.load_gather(rows[pb], [row_vec, col])
                            plsc.store_scatter(tiles[pb], [sdiag[c0] + tbase],
                                               vals)

        def fire_tile_writes(c, pb):
            bt_abs = bt0 + c // _NTG
            t0 = (c % _NTG) * _TG
            for k in range(_TILES_PER_CHUNK):
                tl, dt = k // 4, k % 4
                trow = (t0 + tl) * 512 + dt * 128 + bt_abs
                pltpu.make_async_copy(
                    tiles[pb].at[pl.ds(k * _TILE_ELEMS, _TILE_ELEMS)],
                    out_hbm.at[pl.ds(trow * _TILE_ELEMS, _TILE_ELEMS)],
                    wsem[pb],
                ).start()

        def wait_tile_writes(pb):
            pltpu.make_async_copy(
                tiles[pb],
                out_hbm.at[pl.ds(0, _TILES_PER_CHUNK * _TILE_ELEMS)],
                wsem[pb],
            ).wait()

        build_ord(0, 0)
        start_gather(0)

        @pl.loop(0, _N_CHUNKS // 2)
        def _pair(p):
            for half in range(2):
                c = 2 * p + half
                pb = half
                wait_gather(pb)

                @pl.when(c + 1 < _N_CHUNKS)
                def _prefetch():
                    build_ord(c + 1, 1 - pb)
                    start_gather(1 - pb)

                @pl.when(c >= 2)
                def _free_buf():
                    wait_tile_writes(pb)

                rearrange(pb)
                fire_tile_writes(c, pb)

        wait_tile_writes(0)
        wait_tile_writes(1)

    return gather_kernel


def kernel(token_ids, embeddings):
    idx = token_ids.reshape(-1).astype(jnp.int32)
    table_rm = _make_detile()(embeddings.T.reshape(4, 8, 1_000_000))
    table_rm = table_rm.reshape(1_000_000, _D)
    out = _make_kernel()(idx, table_rm)
    return (out.reshape(_T, 4, 128, 8, 128)
               .transpose(2, 4, 0, 1, 3)
               .reshape(_B, _T, _D))
